# Initial kernel scaffold; baseline (speedup 1.0000x reference)
import jax
import jax.numpy as jnp
from jax.experimental import pallas as pl


def kernel(x, edge_index, Wp, bp, Wr0, br0, Ws0, Wr1, br1, Ws1, Wr2, br2, Ws2, Wr3, br3, Ws3):
    raise NotImplementedError("write your pallas kernel here")


# speedup vs baseline: 10.4041x; 10.4041x over previous
import functools

import jax
import jax.numpy as jnp
from jax import lax
from jax.experimental import pallas as pl
from jax.experimental.pallas import tpu as pltpu
from jax.experimental.pallas import tpu_sc as plsc

N = 10000
E = 320000
D = 128

NC = 2
NS = 16
NW = NC * NS

CH = 80
EPW = E // NW
NCHUNK = EPW // CH
SSTR = 624
LSTR = N - (NS - 1) * SSTR

assert NCHUNK * CH == EPW and NCHUNK % 2 == 1 and (NS - 1) * SSTR + LSTR == N

_mesh = plsc.VectorSubcoreMesh(
    core_axis_name="c", subcore_axis_name="s", num_cores=NC, num_subcores=NS
)


@functools.partial(
    pl.kernel,
    out_type=jax.ShapeDtypeStruct((NC, N, D), jnp.float32),
    mesh=_mesh,
    scratch_types=[
        pltpu.VMEM((EPW,), jnp.int32),
        pltpu.VMEM((CH,), jnp.int32),
        pltpu.VMEM((CH,), jnp.int32),
        pltpu.VMEM((CH, D), jnp.float32),
        pltpu.VMEM((CH, D), jnp.float32),
        pltpu.VMEM_SHARED((N, D), jnp.float32),
        pltpu.SemaphoreType.DMA,
        pltpu.SemaphoreType.DMA,
        pltpu.SemaphoreType.DMA,
        pltpu.SemaphoreType.DMA,
    ],
)
def _sc_segment_sum(h_hbm, src_hbm, dst_hbm, zeros_hbm, out_hbm,
                    src_v, dst_a, dst_b, buf_a, buf_b, agg_sh,
                    sem_ga, sem_gb, sem_da, sem_db):
    c = lax.axis_index("c")
    s = lax.axis_index("s")
    wid = c * NS + s
    ebase = pl.multiple_of(wid * EPW, 8)

    pltpu.sync_copy(src_hbm.at[pl.ds(ebase, EPW)], src_v)

    @pl.when(s < NS - 1)
    def _():
        stripe = pl.ds(pl.multiple_of(s * SSTR, 8), SSTR)
        pltpu.sync_copy(zeros_hbm.at[stripe], agg_sh.at[stripe])

    @pl.when(s == NS - 1)
    def _():
        stripe = pl.ds((NS - 1) * SSTR, LSTR)
        pltpu.sync_copy(zeros_hbm.at[stripe], agg_sh.at[stripe])

    plsc.subcore_barrier()

    def start(j, buf, dbuf, gsem, dsem):
        pltpu.async_copy(h_hbm.at[src_v.at[pl.ds(j * CH, CH)]], buf, gsem)
        pltpu.async_copy(
            dst_hbm.at[pl.ds(pl.multiple_of(ebase + j * CH, 8), CH)], dbuf, dsem)

    def finish(buf, dbuf, gsem, dsem):
        pltpu.make_async_copy(h_hbm.at[src_v.at[pl.ds(0, CH)]], buf, gsem).wait()
        pltpu.make_async_copy(dst_hbm.at[pl.ds(0, CH)], dbuf, dsem).wait()
        pltpu.sync_copy(buf, agg_sh.at[dbuf], add=True)

    start(0, buf_a, dst_a, sem_ga, sem_da)

    @pl.loop(0, NCHUNK - 1, step=2)
    def _(j):
        start(j + 1, buf_b, dst_b, sem_gb, sem_db)
        finish(buf_a, dst_a, sem_ga, sem_da)
        start(j + 2, buf_a, dst_a, sem_ga, sem_da)
        finish(buf_b, dst_b, sem_gb, sem_db)

    finish(buf_a, dst_a, sem_ga, sem_da)

    plsc.subcore_barrier()

    @pl.when(s < NS - 1)
    def _():
        stripe = pl.ds(pl.multiple_of(s * SSTR, 8), SSTR)
        pltpu.sync_copy(agg_sh.at[stripe], out_hbm.at[c, stripe])

    @pl.when(s == NS - 1)
    def _():
        stripe = pl.ds((NS - 1) * SSTR, LSTR)
        pltpu.sync_copy(agg_sh.at[stripe], out_hbm.at[c, stripe])


_BLK = 1000


def _mm_bias_body(x_ref, w_ref, b_ref, o_ref):
    o_ref[...] = (
        jnp.dot(x_ref[...], w_ref[...],
                preferred_element_type=jnp.float32,
                precision=lax.Precision.HIGHEST)
        + b_ref[...]
    )


def _mm_bias(x, wt, b):
    return pl.pallas_call(
        _mm_bias_body,
        out_shape=jax.ShapeDtypeStruct((N, D), jnp.float32),
        grid=(N // _BLK,),
        in_specs=[
            pl.BlockSpec((_BLK, D), lambda i: (i, 0)),
            pl.BlockSpec((D, D), lambda i: (0, 0)),
            pl.BlockSpec((1, D), lambda i: (0, 0)),
        ],
        out_specs=pl.BlockSpec((_BLK, D), lambda i: (i, 0)),
    )(x, wt, b.reshape(1, D))


def _combine_body(agg_a_ref, agg_b_ref, wrt_ref, hw_ref, add_ref, o_ref, *, relu):
    acc = jnp.dot(agg_a_ref[0] + agg_b_ref[0], wrt_ref[...],
                  preferred_element_type=jnp.float32,
                  precision=lax.Precision.HIGHEST)
    acc = acc + hw_ref[...]
    if relu:
        acc = jnp.maximum(acc, 0.0)
    if add_ref is not None:
        acc = acc + add_ref[...]
    o_ref[...] = acc


def _combine(agg, wrt, hw, add):
    has_add = add is not None
    in_specs = [
        pl.BlockSpec((1, _BLK, D), lambda i: (0, i, 0)),
        pl.BlockSpec((1, _BLK, D), lambda i: (1, i, 0)),
        pl.BlockSpec((D, D), lambda i: (0, 0)),
        pl.BlockSpec((_BLK, D), lambda i: (i, 0)),
    ]
    args = [agg, agg, wrt, hw]
    if has_add:
        in_specs.append(pl.BlockSpec((_BLK, D), lambda i: (i, 0)))
        args.append(add)
        body = functools.partial(_combine_body, relu=True)
    else:
        body = lambda a, b, w, h, o: _combine_body(a, b, w, h, None, o, relu=False)
    return pl.pallas_call(
        body,
        out_shape=jax.ShapeDtypeStruct((N, D), jnp.float32),
        grid=(N // _BLK,),
        in_specs=in_specs,
        out_specs=pl.BlockSpec((_BLK, D), lambda i: (i, 0)),
    )(*args)


def kernel(x, edge_index, Wp, bp, Wr0, br0, Ws0, Wr1, br1, Ws1,
           Wr2, br2, Ws2, Wr3, br3, Ws3):
    edge_index = edge_index.astype(jnp.int32)
    src_r = edge_index[0]
    dst_r = edge_index[1]
    zeros = jnp.zeros((N, D), jnp.float32)

    x_res = _mm_bias(x, Wp.T, bp)

    layers = [(Wr0, br0, Ws0), (Wr1, br1, Ws1), (Wr2, br2, Ws2), (Wr3, br3, Ws3)]
    h = x
    for i, (Wr, br, Ws) in enumerate(layers):
        hw = _mm_bias(h, Ws.T, br)
        agg = _sc_segment_sum(h, src_r, dst_r, zeros)
        if i == 0:
            h = _combine(agg, Wr.T, hw, x_res)
        elif i < 3:
            h = _combine(agg, Wr.T, hw, h)
        else:
            h = _combine(agg, Wr.T, hw, None)
    return h


# speedup vs baseline: 11.9448x; 1.1481x over previous
import functools

import jax
import jax.numpy as jnp
from jax import lax
from jax.experimental import pallas as pl
from jax.experimental.pallas import tpu as pltpu
from jax.experimental.pallas import tpu_sc as plsc

N = 10000
E = 320000
D = 128

NC = 2
NS = 16
NW = NC * NS

CH = 80
EPW = E // NW
NCHUNK = EPW // CH
SSTR = 624
LSTR = N - (NS - 1) * SSTR

assert NCHUNK * CH == EPW and NCHUNK % 2 == 1 and (NS - 1) * SSTR + LSTR == N

_mesh = plsc.VectorSubcoreMesh(
    core_axis_name="c", subcore_axis_name="s", num_cores=NC, num_subcores=NS
)


@functools.partial(
    pl.kernel,
    out_type=jax.ShapeDtypeStruct((NC, N, D), jnp.float32),
    mesh=_mesh,
    scratch_types=[
        pltpu.VMEM((EPW,), jnp.int32),
        [pltpu.VMEM((CH,), jnp.int32) for _ in range(3)],
        [pltpu.VMEM((CH, D), jnp.float32) for _ in range(3)],
        pltpu.VMEM_SHARED((N, D), jnp.float32),
        [pltpu.SemaphoreType.DMA for _ in range(3)],
        [pltpu.SemaphoreType.DMA for _ in range(3)],
        [pltpu.SemaphoreType.DMA for _ in range(3)],
    ],
)
def _sc_segment_sum(h_hbm, src_hbm, dst_hbm, zeros_hbm, out_hbm,
                    src_v, dst_bufs, row_bufs, agg_sh,
                    gsems, dsems, ssems):
    c = lax.axis_index("c")
    s = lax.axis_index("s")
    wid = c * NS + s
    ebase = pl.multiple_of(wid * EPW, 8)

    pltpu.sync_copy(src_hbm.at[pl.ds(ebase, EPW)], src_v)

    @pl.when(s < NS - 1)
    def _():
        stripe = pl.ds(pl.multiple_of(s * SSTR, 8), SSTR)
        pltpu.sync_copy(zeros_hbm.at[stripe], agg_sh.at[stripe])

    @pl.when(s == NS - 1)
    def _():
        stripe = pl.ds((NS - 1) * SSTR, LSTR)
        pltpu.sync_copy(zeros_hbm.at[stripe], agg_sh.at[stripe])

    plsc.subcore_barrier()

    def start_g(j, b):
        pltpu.async_copy(h_hbm.at[src_v.at[pl.ds(j * CH, CH)]], row_bufs[b],
                         gsems[b])
        pltpu.async_copy(
            dst_hbm.at[pl.ds(pl.multiple_of(ebase + j * CH, 8), CH)],
            dst_bufs[b], dsems[b])

    def wait_g(b):
        pltpu.make_async_copy(h_hbm.at[src_v.at[pl.ds(0, CH)]], row_bufs[b],
                              gsems[b]).wait()
        pltpu.make_async_copy(dst_hbm.at[pl.ds(0, CH)], dst_bufs[b],
                              dsems[b]).wait()

    def start_s(b):
        pltpu.async_copy(row_bufs[b], agg_sh.at[dst_bufs[b]], ssems[b],
                         add=True)

    def wait_s(b):
        pltpu.make_async_copy(row_bufs[b], agg_sh.at[dst_bufs[b]],
                              ssems[b]).wait()

    start_g(0, 0)
    start_g(1, 1)
    wait_g(0); start_s(0); start_g(2, 2)
    wait_g(1); start_s(1); wait_s(0); start_g(3, 0)
    wait_g(2); start_s(2); wait_s(1); start_g(4, 1)

    @pl.loop(3, NCHUNK - 2, step=3)
    def _(j):
        wait_g(0); start_s(0); wait_s(2); start_g(j + 2, 2)
        wait_g(1); start_s(1); wait_s(0); start_g(j + 3, 0)
        wait_g(2); start_s(2); wait_s(1); start_g(j + 4, 1)

    wait_g(0); start_s(0); wait_s(2)
    wait_g(1); start_s(1); wait_s(0)
    wait_s(1)

    plsc.subcore_barrier()

    @pl.when(s < NS - 1)
    def _():
        stripe = pl.ds(pl.multiple_of(s * SSTR, 8), SSTR)
        pltpu.sync_copy(agg_sh.at[stripe], out_hbm.at[c, stripe])

    @pl.when(s == NS - 1)
    def _():
        stripe = pl.ds((NS - 1) * SSTR, LSTR)
        pltpu.sync_copy(agg_sh.at[stripe], out_hbm.at[c, stripe])


_BLK = 1000


def _mm_bias_body(x_ref, w_ref, b_ref, o_ref):
    o_ref[...] = (
        jnp.dot(x_ref[...], w_ref[...],
                preferred_element_type=jnp.float32,
                precision=lax.Precision.HIGHEST)
        + b_ref[...]
    )


def _mm_bias(x, wt, b):
    return pl.pallas_call(
        _mm_bias_body,
        out_shape=jax.ShapeDtypeStruct((N, D), jnp.float32),
        grid=(N // _BLK,),
        in_specs=[
            pl.BlockSpec((_BLK, D), lambda i: (i, 0)),
            pl.BlockSpec((D, D), lambda i: (0, 0)),
            pl.BlockSpec((1, D), lambda i: (0, 0)),
        ],
        out_specs=pl.BlockSpec((_BLK, D), lambda i: (i, 0)),
    )(x, wt, b.reshape(1, D))


def _combine_body(agg_a_ref, agg_b_ref, wrt_ref, hw_ref, add_ref, o_ref, *, relu):
    acc = jnp.dot(agg_a_ref[0] + agg_b_ref[0], wrt_ref[...],
                  preferred_element_type=jnp.float32,
                  precision=lax.Precision.HIGHEST)
    acc = acc + hw_ref[...]
    if relu:
        acc = jnp.maximum(acc, 0.0)
    if add_ref is not None:
        acc = acc + add_ref[...]
    o_ref[...] = acc


def _combine(agg, wrt, hw, add):
    has_add = add is not None
    in_specs = [
        pl.BlockSpec((1, _BLK, D), lambda i: (0, i, 0)),
        pl.BlockSpec((1, _BLK, D), lambda i: (1, i, 0)),
        pl.BlockSpec((D, D), lambda i: (0, 0)),
        pl.BlockSpec((_BLK, D), lambda i: (i, 0)),
    ]
    args = [agg, agg, wrt, hw]
    if has_add:
        in_specs.append(pl.BlockSpec((_BLK, D), lambda i: (i, 0)))
        args.append(add)
        body = functools.partial(_combine_body, relu=True)
    else:
        body = lambda a, b, w, h, o: _combine_body(a, b, w, h, None, o, relu=False)
    return pl.pallas_call(
        body,
        out_shape=jax.ShapeDtypeStruct((N, D), jnp.float32),
        grid=(N // _BLK,),
        in_specs=in_specs,
        out_specs=pl.BlockSpec((_BLK, D), lambda i: (i, 0)),
    )(*args)


def kernel(x, edge_index, Wp, bp, Wr0, br0, Ws0, Wr1, br1, Ws1,
           Wr2, br2, Ws2, Wr3, br3, Ws3):
    edge_index = edge_index.astype(jnp.int32)
    src_r = edge_index[0]
    dst_r = edge_index[1]
    zeros = jnp.zeros((N, D), jnp.float32)

    x_res = _mm_bias(x, Wp.T, bp)

    layers = [(Wr0, br0, Ws0), (Wr1, br1, Ws1), (Wr2, br2, Ws2), (Wr3, br3, Ws3)]
    h = x
    for i, (Wr, br, Ws) in enumerate(layers):
        hw = _mm_bias(h, Ws.T, br)
        agg = _sc_segment_sum(h, src_r, dst_r, zeros)
        if i == 0:
            h = _combine(agg, Wr.T, hw, x_res)
        elif i < 3:
            h = _combine(agg, Wr.T, hw, h)
        else:
            h = _combine(agg, Wr.T, hw, None)
    return h
